# Initial kernel scaffold; baseline (speedup 1.0000x reference)
import jax
import jax.numpy as jnp
from jax.experimental import pallas as pl


def kernel(x, norm_scale, norm_bias, gate_w, W1, B1, W2, B2):
    raise NotImplementedError("write your pallas kernel here")


# speedup vs baseline: 3.6143x; 3.6143x over previous
import jax
import jax.numpy as jnp
from jax import lax
from jax.experimental import pallas as pl

B, L, D = 1, 2048, 768
E, K, H = 8, 2, 1536
N = B * L


def _gate_kernel(x_ref, scale_ref, bias_ref, gw_ref, xn_ref, gates_ref, bal_ref):
    x = x_ref[...]
    mu = jnp.mean(x, axis=1, keepdims=True)
    var = jnp.mean((x - mu) ** 2, axis=1, keepdims=True)
    xn = (x - mu) / jnp.sqrt(var + 1e-5) * scale_ref[...] + bias_ref[...]
    xn_ref[...] = xn
    logits = lax.dot_general(gw_ref[...], xn, (((1,), (1,)), ((), ())),
                             preferred_element_type=jnp.float32)
    iota_e = lax.broadcasted_iota(jnp.int32, (E, N), 0)
    m1 = jnp.max(logits, axis=0, keepdims=True)
    i1 = jnp.min(jnp.where(logits == m1, iota_e, E), axis=0, keepdims=True)
    masked = jnp.where(iota_e == i1, -jnp.inf, logits)
    m2 = jnp.max(masked, axis=0, keepdims=True)
    i2 = jnp.min(jnp.where(masked == m2, iota_e, E), axis=0, keepdims=True)
    e2 = jnp.exp(m2 - m1)
    denom = 1.0 + e2
    g1 = 1.0 / denom
    g2 = e2 / denom
    gates = (jnp.where(iota_e == i1, g1, 0.0)
             + jnp.where(iota_e == i2, g2, 0.0))
    gates_ref[...] = gates
    load = jnp.mean(gates, axis=1, keepdims=True)
    bal_ref[...] = jnp.mean((load - 1.0 / E) ** 2).reshape(1, 1)


def _expert_kernel(x_ref, xn_ref, gates_ref, w1_ref, b1_ref, w2_ref, b2_ref,
                   out_ref):
    e = pl.program_id(0)
    xn = xn_ref[...]
    w1 = w1_ref[0]
    w2 = w2_ref[0]
    h = lax.dot_general(xn, w1, (((1,), (1,)), ((), ())),
                        preferred_element_type=jnp.float32) + b1_ref[0]
    h = 0.5 * h * (1.0 + lax.erf(h * 0.7071067811865476))
    oe = lax.dot_general(h, w2, (((1,), (1,)), ((), ())),
                         preferred_element_type=jnp.float32) + b2_ref[0]
    iota_e = lax.broadcasted_iota(jnp.int32, (E, N), 0)
    g = jnp.sum(jnp.where(iota_e == e, gates_ref[...], 0.0), axis=0)[:, None]
    contrib = g * oe

    @pl.when(e == 0)
    def _():
        out_ref[...] = x_ref[...] + contrib

    @pl.when(e != 0)
    def _():
        out_ref[...] = out_ref[...] + contrib


def kernel(x, norm_scale, norm_bias, gate_w, W1, B1, W2, B2):
    xf = x.reshape(N, D)
    xn, gates, bal = pl.pallas_call(
        _gate_kernel,
        out_shape=[
            jax.ShapeDtypeStruct((N, D), jnp.float32),
            jax.ShapeDtypeStruct((E, N), jnp.float32),
            jax.ShapeDtypeStruct((1, 1), jnp.float32),
        ],
    )(xf, norm_scale.reshape(1, D), norm_bias.reshape(1, D), gate_w)

    out = pl.pallas_call(
        _expert_kernel,
        grid=(E,),
        in_specs=[
            pl.BlockSpec((N, D), lambda e: (0, 0)),
            pl.BlockSpec((N, D), lambda e: (0, 0)),
            pl.BlockSpec((E, N), lambda e: (0, 0)),
            pl.BlockSpec((1, H, D), lambda e: (e, 0, 0)),
            pl.BlockSpec((1, 1, H), lambda e: (e, 0, 0)),
            pl.BlockSpec((1, D, H), lambda e: (e, 0, 0)),
            pl.BlockSpec((1, 1, D), lambda e: (e, 0, 0)),
        ],
        out_specs=pl.BlockSpec((N, D), lambda e: (0, 0)),
        out_shape=jax.ShapeDtypeStruct((N, D), jnp.float32),
    )(xf, xn, gates, W1, B1.reshape(E, 1, H), W2, B2.reshape(E, 1, D))

    return out.reshape(B, L, D), bal.reshape(())
